# Initial kernel scaffold; baseline (speedup 1.0000x reference)
import jax
import jax.numpy as jnp
from jax.experimental import pallas as pl


def kernel(x):
    raise NotImplementedError("write your pallas kernel here")


# speedup vs baseline: 11.5248x; 11.5248x over previous
import functools

import jax
import jax.numpy as jnp
from jax.experimental import pallas as pl

_N = 4096
_BR = 256
_KNN = 16
_MAX_EDGE = 2.0


def _d2_block(xi, xT):
    sqi = jnp.sum(xi * xi, axis=1, keepdims=True)
    sqj = jnp.sum(xT * xT, axis=0, keepdims=True)
    cross = jnp.dot(xi, xT, preferred_element_type=jnp.float32)
    return jnp.maximum(sqi + sqj - 2.0 * cross, 0.0)


def _dtm_kernel(xi_ref, xT_ref, dtm_ref):
    d2 = _d2_block(xi_ref[...], xT_ref[...])
    t = jnp.full((_BR, 1), -jnp.inf, dtype=jnp.float32)
    c = jnp.zeros((_BR, 1), dtype=jnp.float32)
    for _ in range(_KNN):
        active = c < float(_KNN)
        masked = jnp.where(d2 > t, d2, jnp.inf)
        newt = jnp.min(masked, axis=1, keepdims=True)
        t = jnp.where(active, newt, t)
        c = jnp.sum((d2 <= t).astype(jnp.float32), axis=1, keepdims=True)
    below = d2 < t
    s = jnp.sum(jnp.where(below, d2, 0.0), axis=1, keepdims=True)
    cnt = jnp.sum(below.astype(jnp.float32), axis=1, keepdims=True)
    s = s + t * (float(_KNN) - cnt)
    dtm_ref[...] = jnp.sqrt(s / float(_KNN))


def _edge_kernel(xi_ref, xT_ref, fi_ref, fjT_ref, out_ref):
    d2 = _d2_block(xi_ref[...], xT_ref[...])
    dist = jnp.sqrt(jnp.maximum(d2, 1e-12))
    fi = fi_ref[...]
    fj = fjT_ref[...]
    fmax = jnp.maximum(fi, fj)
    edge = jnp.where(dist <= jnp.abs(fi - fj), fmax, (fi + fj + dist) * 0.5)
    out_ref[...] = jnp.minimum(edge, _MAX_EDGE)


@functools.partial(jax.jit)
def kernel(x):
    xT = x.T
    nblk = _N // _BR
    dtm = pl.pallas_call(
        _dtm_kernel,
        grid=(nblk,),
        in_specs=[
            pl.BlockSpec((_BR, 3), lambda i: (i, 0)),
            pl.BlockSpec((3, _N), lambda i: (0, 0)),
        ],
        out_specs=pl.BlockSpec((_BR, 1), lambda i: (i, 0)),
        out_shape=jax.ShapeDtypeStruct((_N, 1), jnp.float32),
    )(x, xT)
    dtmT = dtm.reshape(1, _N)
    edge = pl.pallas_call(
        _edge_kernel,
        grid=(nblk,),
        in_specs=[
            pl.BlockSpec((_BR, 3), lambda i: (i, 0)),
            pl.BlockSpec((3, _N), lambda i: (0, 0)),
            pl.BlockSpec((_BR, 1), lambda i: (i, 0)),
            pl.BlockSpec((1, _N), lambda i: (0, 0)),
        ],
        out_specs=pl.BlockSpec((_BR, _N), lambda i: (i, 0)),
        out_shape=jax.ShapeDtypeStruct((_N, _N), jnp.float32),
    )(x, xT, dtm, dtmT)
    return edge


# speedup vs baseline: 17.4529x; 1.5144x over previous
import functools

import jax
import jax.numpy as jnp
from jax.experimental import pallas as pl

_N = 4096
_BR = 256
_KNN = 16
_MAX_EDGE = 2.0


def _d2_block(xi, xT):
    sqi = jnp.sum(xi * xi, axis=1, keepdims=True)
    sqj = jnp.sum(xT * xT, axis=0, keepdims=True)
    cross = jnp.dot(xi, xT, preferred_element_type=jnp.float32)
    return jnp.maximum(sqi + sqj - 2.0 * cross, 0.0)


def _dtm_kernel(xi_ref, xT_ref, dtm_ref):
    d2 = _d2_block(xi_ref[...], xT_ref[...])
    g = jnp.minimum(d2[:, : _N // 2], d2[:, _N // 2 :])
    g = jnp.minimum(g[:, : _N // 4], g[:, _N // 4 :])
    g = jnp.minimum(g[:, : _N // 8], g[:, _N // 8 :])
    t = jnp.full((_BR, 1), -jnp.inf, dtype=jnp.float32)
    c = jnp.zeros((_BR, 1), dtype=jnp.float32)
    for _ in range(_KNN):
        active = c < float(_KNN)
        newt = jnp.min(jnp.where(g > t, g, jnp.inf), axis=1, keepdims=True)
        t = jnp.where(active, newt, t)
        c = jnp.sum((g <= t).astype(jnp.float32), axis=1, keepdims=True)
    in_c = d2 <= t
    n = jnp.sum(in_c.astype(jnp.float32), axis=1, keepdims=True)
    s = jnp.sum(jnp.where(in_c, d2, 0.0), axis=1, keepdims=True)
    k = n - float(_KNN)
    cm = jnp.where(in_c, d2, -jnp.inf)
    zeros = jnp.zeros((_BR, 1), dtype=jnp.float32)
    nmax = jnp.max(k).astype(jnp.int32)

    def _drop(_, carry):
        cm, removed, s, lastmx = carry
        need = removed < k
        mx = jnp.max(cm, axis=1, keepdims=True)
        eq = (cm == mx) & need
        cnt = jnp.sum(eq.astype(jnp.float32), axis=1, keepdims=True)
        cm = jnp.where(eq, -jnp.inf, cm)
        removed = removed + cnt
        s = s - jnp.where(need, cnt * mx, 0.0)
        lastmx = jnp.where(need, mx, lastmx)
        return cm, removed, s, lastmx

    cm, removed, s, lastmx = jax.lax.fori_loop(
        0, nmax, _drop, (cm, zeros, s, zeros))
    s = s + jnp.maximum(removed - k, 0.0) * lastmx
    dtm_ref[...] = jnp.sqrt(s * (1.0 / float(_KNN)))


def _edge_kernel(xi_ref, xT_ref, fi_ref, fjT_ref, out_ref):
    d2 = _d2_block(xi_ref[...], xT_ref[...])
    dist = jnp.sqrt(jnp.maximum(d2, 1e-12))
    fi = fi_ref[...]
    fj = fjT_ref[...]
    fmax = jnp.maximum(fi, fj)
    edge = jnp.where(dist <= jnp.abs(fi - fj), fmax, (fi + fj + dist) * 0.5)
    out_ref[...] = jnp.minimum(edge, _MAX_EDGE)


@functools.partial(jax.jit)
def kernel(x):
    xT = x.T
    nblk = _N // _BR
    dtm = pl.pallas_call(
        _dtm_kernel,
        grid=(nblk,),
        in_specs=[
            pl.BlockSpec((_BR, 3), lambda i: (i, 0)),
            pl.BlockSpec((3, _N), lambda i: (0, 0)),
        ],
        out_specs=pl.BlockSpec((_BR, 1), lambda i: (i, 0)),
        out_shape=jax.ShapeDtypeStruct((_N, 1), jnp.float32),
    )(x, xT)
    dtmT = dtm.reshape(1, _N)
    edge = pl.pallas_call(
        _edge_kernel,
        grid=(nblk,),
        in_specs=[
            pl.BlockSpec((_BR, 3), lambda i: (i, 0)),
            pl.BlockSpec((3, _N), lambda i: (0, 0)),
            pl.BlockSpec((_BR, 1), lambda i: (i, 0)),
            pl.BlockSpec((1, _N), lambda i: (0, 0)),
        ],
        out_specs=pl.BlockSpec((_BR, _N), lambda i: (i, 0)),
        out_shape=jax.ShapeDtypeStruct((_N, _N), jnp.float32),
    )(x, xT, dtm, dtmT)
    return edge
